# baseline (speedup 1.0000x reference)
import functools

import jax
import jax.numpy as jnp
from jax import lax
from jax.experimental import pallas as pl
from jax.experimental.pallas import tpu as pltpu
from jax.experimental.pallas import tpu_sc as plsc

N_DRUGS = 5000
N_NODES = 10000
N_PAD = 10240
EMB = 128
NC = 2
NS = 16
NW = NC * NS
CL = 120
VCL = 128
NBUF = 3
NIDX = 6


def _make_spmm(nchunk):
    assert nchunk % NIDX == 0
    mesh = plsc.VectorSubcoreMesh(core_axis_name="c", subcore_axis_name="s")
    rows_per_tile = N_PAD // NS
    wb = 80

    @functools.partial(
        pl.kernel,
        mesh=mesh,
        out_type=jax.ShapeDtypeStruct((NC, N_PAD, EMB), jnp.float32),
        scratch_types=[
            pltpu.VMEM((NIDX, CL), jnp.int32),
            pltpu.VMEM((NIDX, CL), jnp.int32),
            pltpu.VMEM((NIDX, VCL), jnp.float32),
            pltpu.VMEM((NBUF, CL, EMB), jnp.float32),
            pltpu.VMEM_SHARED((N_PAD, EMB), jnp.float32),
        ]
        + [pltpu.SemaphoreType.DMA] * (NIDX + NBUF + NBUF),
    )
    def spmm(x_hbm, row_hbm, col_hbm, val_hbm, out_hbm,
             row_v, col_v, val_v, gath_v, acc_sh, *allsems):
        cid = lax.axis_index("c")
        sid = lax.axis_index("s")
        wid = sid * NC + cid
        semi = allsems[0:NIDX]
        semg = allsems[NIDX:NIDX + NBUF]
        sems = allsems[NIDX + NBUF:]
        base = wid * (nchunk * CL)
        vbase = wid * (nchunk * VCL)

        def start_idx(j, i):
            src = pl.ds(base + j * CL, CL)
            pltpu.async_copy(row_hbm.at[src], row_v.at[i], semi[i])
            pltpu.async_copy(col_hbm.at[src], col_v.at[i], semi[i])
            pltpu.async_copy(val_hbm.at[pl.ds(vbase + j * VCL, VCL)],
                             val_v.at[i], semi[i])

        def wait_idx(i):
            src = pl.ds(base, CL)
            pltpu.make_async_copy(row_hbm.at[src], row_v.at[i], semi[i]).wait()
            pltpu.make_async_copy(col_hbm.at[src], col_v.at[i], semi[i]).wait()
            pltpu.make_async_copy(val_hbm.at[pl.ds(vbase, VCL)],
                                  val_v.at[i], semi[i]).wait()

        def start_gather(b, i):
            pltpu.async_copy(x_hbm.at[col_v.at[i]], gath_v.at[b], semg[b])

        def wait_gather(b):
            pltpu.make_async_copy(
                x_hbm.at[col_v.at[0]], gath_v.at[b], semg[b]).wait()

        def start_scatter(b, i):
            pltpu.async_copy(gath_v.at[b], acc_sh.at[row_v.at[i]],
                             sems[b], add=True)

        def wait_scatter(b):
            pltpu.make_async_copy(
                gath_v.at[b], acc_sh.at[row_v.at[0]], sems[b]).wait()

        def scale(b, i):
            def grp(g, _):
                vals16 = val_v[i, pl.ds(g * 8, 16)]
                for k in range(8):
                    s = vals16[k]
                    e = g * 8 + k
                    for h in range(EMB // 16):
                        sl = pl.ds(h * 16, 16)
                        gath_v[b, e, sl] = gath_v[b, e, sl] * s
                return 0

            lax.fori_loop(0, CL // 8, grp, 0)

        def step(j, b, i6, first=False):
            p = (b + 2) % NBUF
            ig = (i6 + 2) % NIDX
            isn = (i6 + 3) % NIDX
            wait_gather(b)
            scale(b, i6)
            start_scatter(b, i6)
            if not first:
                wait_scatter(p)
            wait_idx(ig)
            start_gather(p, ig)
            jn = j + NBUF
            jn = jnp.where(jn >= nchunk, jn - nchunk, jn)
            start_idx(jn, isn)

        start_idx(0, 0)
        start_idx(jnp.int32(1), 1)
        start_idx(jnp.int32(2), 2)

        zeros16 = jnp.zeros((16,), jnp.float32)

        def zrow(r, _):
            for h in range(EMB // 16):
                gath_v[0, r, pl.ds(h * 16, 16)] = zeros16
            return 0

        lax.fori_loop(0, wb, zrow, 0)

        def zacc(k, _):
            start = sid * rows_per_tile + k * wb
            pltpu.sync_copy(gath_v.at[0, pl.ds(0, wb)],
                            acc_sh.at[pl.ds(start, wb)])
            return 0

        lax.fori_loop(0, rows_per_tile // wb, zacc, 0)
        plsc.subcore_barrier()

        wait_idx(0)
        start_gather(0, 0)
        wait_idx(1)
        start_gather(1, 1)

        step(jnp.int32(0), 0, 0, first=True)
        for t in range(1, NIDX):
            step(jnp.int32(t), t % NBUF, t)

        def iter_body(it, _):
            j = NIDX + NIDX * it
            for t in range(NIDX):
                step(j + t, t % NBUF, t)
            return 0

        lax.fori_loop(0, nchunk // NIDX - 1, iter_body, 0)

        wait_scatter((nchunk - 1) % NBUF)
        wait_gather(nchunk % NBUF)
        wait_gather((nchunk + 1) % NBUF)
        wait_idx((nchunk + 2) % NIDX)
        plsc.subcore_barrier()

        def wback(k, _):
            start = sid * rows_per_tile + k * wb
            pltpu.sync_copy(acc_sh.at[pl.ds(start, wb)],
                            gath_v.at[0, pl.ds(0, wb)])
            pltpu.sync_copy(gath_v.at[0, pl.ds(0, wb)],
                            out_hbm.at[cid].at[pl.ds(start, wb)])
            return 0

        lax.fori_loop(0, rows_per_tile // wb, wback, 0)

    return spmm


def kernel(drug_emb, disease_emb, adj_indices, adj_values):
    nnz = adj_values.shape[0]
    nchunk = -(-nnz // (NW * CL))
    nchunk += (-nchunk) % NIDX
    total = NW * nchunk * CL
    pad = total - nnz

    row = adj_indices[0].astype(jnp.int32)
    col = adj_indices[1].astype(jnp.int32)
    val = adj_values.astype(jnp.float32)
    if pad:
        fill = (jnp.arange(pad, dtype=jnp.int32) * 37) % N_NODES
        row = jnp.concatenate([row, fill])
        col = jnp.concatenate([col, fill])
        val = jnp.concatenate([val, jnp.zeros((pad,), jnp.float32)])
    val = jnp.pad(val.reshape(NW * nchunk, CL), ((0, 0), (0, VCL - CL))
                  ).reshape(-1)

    spmm = _make_spmm(nchunk)

    x = jnp.concatenate([drug_emb, disease_emb], axis=0)
    acc = x
    for _ in range(3):
        p = spmm(x, row, col, val)
        x = (p[0] + p[1])[:N_NODES]
        acc = acc + x
    out = acc * 0.25
    return (out[:N_DRUGS], out[N_DRUGS:])


# speedup vs baseline: 2.0569x; 2.0569x over previous
import functools

import jax
import jax.numpy as jnp
from jax import lax
from jax.experimental import pallas as pl
from jax.experimental.pallas import tpu as pltpu
from jax.experimental.pallas import tpu_sc as plsc

N_DRUGS = 5000
N_NODES = 10000
N_PAD = 10240
EMB = 128
NC = 2
NS = 16
NW = NC * NS
CL = 120
VCL = 128
NBUF = 3
NIDX = 6


def _make_spmm(nchunk):
    assert nchunk % NIDX == 0
    mesh = plsc.VectorSubcoreMesh(core_axis_name="c", subcore_axis_name="s")
    rows_per_tile = N_PAD // NS
    wb = 80

    @functools.partial(
        pl.kernel,
        mesh=mesh,
        out_type=jax.ShapeDtypeStruct((NC, N_PAD, EMB), jnp.float32),
        scratch_types=[
            pltpu.VMEM((NIDX, CL), jnp.int32),
            pltpu.VMEM((NIDX, CL), jnp.int32),
            pltpu.VMEM((NIDX, VCL), jnp.float32),
            pltpu.VMEM((NBUF, CL, EMB), jnp.float32),
            pltpu.VMEM_SHARED((N_PAD, EMB), jnp.float32),
        ]
        + [pltpu.SemaphoreType.DMA] * (NIDX + NBUF + NBUF),
    )
    def spmm(x_hbm, row_hbm, col_hbm, val_hbm, out_hbm,
             row_v, col_v, val_v, gath_v, acc_sh, *allsems):
        cid = lax.axis_index("c")
        sid = lax.axis_index("s")
        wid = sid * NC + cid
        semi = allsems[0:NIDX]
        semg = allsems[NIDX:NIDX + NBUF]
        sems = allsems[NIDX + NBUF:]
        base = wid * (nchunk * CL)
        vbase = wid * (nchunk * VCL)

        def start_idx(j, i):
            src = pl.ds(base + j * CL, CL)
            pltpu.async_copy(row_hbm.at[src], row_v.at[i], semi[i])
            pltpu.async_copy(col_hbm.at[src], col_v.at[i], semi[i])
            pltpu.async_copy(val_hbm.at[pl.ds(vbase + j * VCL, VCL)],
                             val_v.at[i], semi[i])

        def wait_idx(i):
            src = pl.ds(base, CL)
            pltpu.make_async_copy(row_hbm.at[src], row_v.at[i], semi[i]).wait()
            pltpu.make_async_copy(col_hbm.at[src], col_v.at[i], semi[i]).wait()
            pltpu.make_async_copy(val_hbm.at[pl.ds(vbase, VCL)],
                                  val_v.at[i], semi[i]).wait()

        def start_gather(b, i):
            pltpu.async_copy(x_hbm.at[col_v.at[i]], gath_v.at[b], semg[b])

        def wait_gather(b):
            pltpu.make_async_copy(
                x_hbm.at[col_v.at[0]], gath_v.at[b], semg[b]).wait()

        def start_scatter(b, i):
            pltpu.async_copy(gath_v.at[b], acc_sh.at[row_v.at[i]],
                             sems[b], add=True)

        def wait_scatter(b):
            pltpu.make_async_copy(
                gath_v.at[b], acc_sh.at[row_v.at[0]], sems[b]).wait()

        def scale(b, i):
            def grp(g, _):
                vals16 = val_v[i, pl.ds(g * 16, 16)]
                for k in range(16):
                    s = vals16[k]
                    e = g * 16 + k
                    for h in range(EMB // 16):
                        sl = pl.ds(h * 16, 16)
                        gath_v[b, e, sl] = gath_v[b, e, sl] * s
                return 0

            lax.fori_loop(0, CL // 16, grp, 0)
            vals16 = val_v[i, pl.ds(112, 16)]
            for k in range(8):
                s = vals16[k]
                e = 112 + k
                for h in range(EMB // 16):
                    sl = pl.ds(h * 16, 16)
                    gath_v[b, e, sl] = gath_v[b, e, sl] * s

        def step(j, b, i6, first=False):
            p = (b + 2) % NBUF
            ig = (i6 + 2) % NIDX
            isn = (i6 + 3) % NIDX
            wait_gather(b)
            scale(b, i6)
            start_scatter(b, i6)
            if not first:
                wait_scatter(p)
            wait_idx(ig)
            start_gather(p, ig)
            jn = j + NBUF
            jn = jnp.where(jn >= nchunk, jn - nchunk, jn)
            start_idx(jn, isn)

        start_idx(0, 0)
        start_idx(jnp.int32(1), 1)
        start_idx(jnp.int32(2), 2)

        zeros16 = jnp.zeros((16,), jnp.float32)

        def zrow(r, _):
            for h in range(EMB // 16):
                gath_v[0, r, pl.ds(h * 16, 16)] = zeros16
            return 0

        lax.fori_loop(0, wb, zrow, 0)

        def zacc(k, _):
            start = sid * rows_per_tile + k * wb
            pltpu.sync_copy(gath_v.at[0, pl.ds(0, wb)],
                            acc_sh.at[pl.ds(start, wb)])
            return 0

        lax.fori_loop(0, rows_per_tile // wb, zacc, 0)
        plsc.subcore_barrier()

        wait_idx(0)
        start_gather(0, 0)
        wait_idx(1)
        start_gather(1, 1)

        step(jnp.int32(0), 0, 0, first=True)
        for t in range(1, NIDX):
            step(jnp.int32(t), t % NBUF, t)

        def iter_body(it, _):
            j = NIDX + NIDX * it
            for t in range(NIDX):
                step(j + t, t % NBUF, t)
            return 0

        lax.fori_loop(0, nchunk // NIDX - 1, iter_body, 0)

        wait_scatter((nchunk - 1) % NBUF)
        wait_gather(nchunk % NBUF)
        wait_gather((nchunk + 1) % NBUF)
        wait_idx((nchunk + 2) % NIDX)
        plsc.subcore_barrier()

        def wback(k, _):
            start = sid * rows_per_tile + k * wb
            pltpu.sync_copy(acc_sh.at[pl.ds(start, wb)],
                            gath_v.at[0, pl.ds(0, wb)])
            pltpu.sync_copy(gath_v.at[0, pl.ds(0, wb)],
                            out_hbm.at[cid].at[pl.ds(start, wb)])
            return 0

        lax.fori_loop(0, rows_per_tile // wb, wback, 0)

    return spmm


def kernel(drug_emb, disease_emb, adj_indices, adj_values):
    nnz = adj_values.shape[0]
    nchunk = -(-nnz // (NW * CL))
    nchunk += (-nchunk) % NIDX
    total = NW * nchunk * CL
    pad = total - nnz

    row = adj_indices[0].astype(jnp.int32)
    col = adj_indices[1].astype(jnp.int32)
    val = adj_values.astype(jnp.float32)
    if pad:
        fill = (jnp.arange(pad, dtype=jnp.int32) * 37) % N_NODES
        row = jnp.concatenate([row, fill])
        col = jnp.concatenate([col, fill])
        val = jnp.concatenate([val, jnp.zeros((pad,), jnp.float32)])
    val = jnp.pad(val.reshape(NW * nchunk, CL), ((0, 0), (0, VCL - CL))
                  ).reshape(-1)

    spmm = _make_spmm(nchunk)

    x = jnp.concatenate([drug_emb, disease_emb], axis=0)
    acc = x
    for _ in range(3):
        p = spmm(x, row, col, val)
        x = (p[0] + p[1])[:N_NODES]
        acc = acc + x
    out = acc * 0.25
    return (out[:N_DRUGS], out[N_DRUGS:])


# speedup vs baseline: 2.1278x; 1.0345x over previous
import functools

import jax
import jax.numpy as jnp
from jax import lax
from jax.experimental import pallas as pl
from jax.experimental.pallas import tpu as pltpu
from jax.experimental.pallas import tpu_sc as plsc

N_DRUGS = 5000
N_NODES = 10000
N_PAD = 10240
EMB = 128
NC = 2
NS = 16
NW = NC * NS
CL = 112
NBUF = 3
NIDX = 6


def _make_spmm(nchunk):
    assert nchunk % NIDX == 0
    mesh = plsc.VectorSubcoreMesh(core_axis_name="c", subcore_axis_name="s")
    rows_per_tile = N_PAD // NS
    wb = 80

    @functools.partial(
        pl.kernel,
        mesh=mesh,
        out_type=jax.ShapeDtypeStruct((NC, N_PAD, EMB), jnp.float32),
        scratch_types=[
            pltpu.VMEM((NIDX, CL), jnp.int32),
            pltpu.VMEM((NIDX, CL), jnp.int32),
            pltpu.VMEM((NIDX, CL), jnp.float32),
            pltpu.VMEM((NBUF, CL, EMB), jnp.float32),
            pltpu.VMEM_SHARED((N_PAD, EMB), jnp.float32),
        ]
        + [pltpu.SemaphoreType.DMA] * (NIDX + NBUF + NBUF),
    )
    def spmm(x_hbm, row_hbm, col_hbm, val_hbm, out_hbm,
             row_v, col_v, val_v, gath_v, acc_sh, *allsems):
        cid = lax.axis_index("c")
        sid = lax.axis_index("s")
        wid = sid * NC + cid
        semi = allsems[0:NIDX]
        semg = allsems[NIDX:NIDX + NBUF]
        sems = allsems[NIDX + NBUF:]
        base = wid * (nchunk * CL)

        def start_idx(j, i):
            src = pl.ds(base + j * CL, CL)
            pltpu.async_copy(row_hbm.at[src], row_v.at[i], semi[i])
            pltpu.async_copy(col_hbm.at[src], col_v.at[i], semi[i])
            pltpu.async_copy(val_hbm.at[src], val_v.at[i], semi[i])

        def wait_idx(i):
            src = pl.ds(base, CL)
            pltpu.make_async_copy(row_hbm.at[src], row_v.at[i], semi[i]).wait()
            pltpu.make_async_copy(col_hbm.at[src], col_v.at[i], semi[i]).wait()
            pltpu.make_async_copy(val_hbm.at[src], val_v.at[i], semi[i]).wait()

        def start_gather(b, i):
            pltpu.async_copy(x_hbm.at[col_v.at[i]], gath_v.at[b], semg[b])

        def wait_gather(b):
            pltpu.make_async_copy(
                x_hbm.at[col_v.at[0]], gath_v.at[b], semg[b]).wait()

        def start_scatter(b, i):
            pltpu.async_copy(gath_v.at[b], acc_sh.at[row_v.at[i]],
                             sems[b], add=True)

        def wait_scatter(b):
            pltpu.make_async_copy(
                gath_v.at[b], acc_sh.at[row_v.at[0]], sems[b]).wait()

        def scale(b, i):
            def grp(g, _):
                vals16 = val_v[i, pl.ds(g * 16, 16)]
                for k in range(16):
                    s = vals16[k]
                    e = g * 16 + k
                    for h in range(EMB // 16):
                        sl = pl.ds(h * 16, 16)
                        gath_v[b, e, sl] = gath_v[b, e, sl] * s
                return 0

            lax.fori_loop(0, CL // 16, grp, 0)

        def step(j, b, i6, first=False):
            p = (b + 2) % NBUF
            ig = (i6 + 2) % NIDX
            isn = (i6 + 3) % NIDX
            wait_gather(b)
            scale(b, i6)
            start_scatter(b, i6)
            if not first:
                wait_scatter(p)
            wait_idx(ig)
            start_gather(p, ig)
            jn = j + NBUF
            jn = jnp.where(jn >= nchunk, jn - nchunk, jn)
            start_idx(jn, isn)

        start_idx(0, 0)
        start_idx(jnp.int32(1), 1)
        start_idx(jnp.int32(2), 2)

        zeros16 = jnp.zeros((16,), jnp.float32)

        def zrow(r, _):
            for h in range(EMB // 16):
                gath_v[0, r, pl.ds(h * 16, 16)] = zeros16
            return 0

        lax.fori_loop(0, wb, zrow, 0)

        def zacc(k, _):
            start = sid * rows_per_tile + k * wb
            pltpu.async_copy(gath_v.at[0, pl.ds(0, wb)],
                             acc_sh.at[pl.ds(start, wb)], semg[0])
            return 0

        lax.fori_loop(0, rows_per_tile // wb, zacc, 0)

        def zaccw(k, _):
            pltpu.make_async_copy(gath_v.at[0, pl.ds(0, wb)],
                                  acc_sh.at[pl.ds(0, wb)], semg[0]).wait()
            return 0

        lax.fori_loop(0, rows_per_tile // wb, zaccw, 0)
        plsc.subcore_barrier()

        wait_idx(0)
        start_gather(0, 0)
        wait_idx(1)
        start_gather(1, 1)

        step(jnp.int32(0), 0, 0, first=True)
        for t in range(1, NIDX):
            step(jnp.int32(t), t % NBUF, t)

        def iter_body(it, _):
            j = NIDX + NIDX * it
            for t in range(NIDX):
                step(j + t, t % NBUF, t)
            return 0

        lax.fori_loop(0, nchunk // NIDX - 1, iter_body, 0)

        wait_scatter((nchunk - 1) % NBUF)
        wait_gather(nchunk % NBUF)
        wait_gather((nchunk + 1) % NBUF)
        wait_idx((nchunk + 2) % NIDX)
        plsc.subcore_barrier()

        nwb = rows_per_tile // wb
        for k in range(nwb):
            b = k % NBUF
            start = sid * rows_per_tile + k * wb
            if k >= NBUF:
                pltpu.make_async_copy(
                    gath_v.at[b, pl.ds(0, wb)],
                    out_hbm.at[cid].at[pl.ds(0, wb)], semg[b]).wait()
            pltpu.sync_copy(acc_sh.at[pl.ds(start, wb)],
                            gath_v.at[b, pl.ds(0, wb)])
            pltpu.async_copy(gath_v.at[b, pl.ds(0, wb)],
                             out_hbm.at[cid].at[pl.ds(start, wb)], semg[b])
        for k in range(nwb - NBUF, nwb):
            b = k % NBUF
            pltpu.make_async_copy(
                gath_v.at[b, pl.ds(0, wb)],
                out_hbm.at[cid].at[pl.ds(0, wb)], semg[b]).wait()

    return spmm


def kernel(drug_emb, disease_emb, adj_indices, adj_values):
    nnz = adj_values.shape[0]
    nchunk = -(-nnz // (NW * CL))
    nchunk += (-nchunk) % NIDX
    total = NW * nchunk * CL
    pad = total - nnz

    row = adj_indices[0].astype(jnp.int32)
    col = adj_indices[1].astype(jnp.int32)
    val = adj_values.astype(jnp.float32)
    if pad:
        fill = (jnp.arange(pad, dtype=jnp.int32) * 37) % N_NODES
        row = jnp.concatenate([row, fill])
        col = jnp.concatenate([col, fill])
        val = jnp.concatenate([val, jnp.zeros((pad,), jnp.float32)])

    spmm = _make_spmm(nchunk)

    x = jnp.concatenate([drug_emb, disease_emb], axis=0)
    acc = x
    for _ in range(3):
        p = spmm(x, row, col, val)
        x = (p[0] + p[1])[:N_NODES]
        acc = acc + x
    out = acc * 0.25
    return (out[:N_DRUGS], out[N_DRUGS:])
